# initial kernel scaffold (unmeasured)
import functools

import jax
import jax.numpy as jnp
from jax import lax
from jax.experimental import pallas as pl
from jax.experimental.pallas import tpu as pltpu

B, S, H, Dh, Dr = 4, 256, 32, 128, 64
D = 4096
DC = 256
M = B * S


def _mm_body(a_ref, b_ref, o_ref):
    o_ref[...] = jnp.dot(a_ref[...], b_ref[...],
                         preferred_element_type=jnp.float32)


def _matmul(a, b, block_n):
    m, k = a.shape
    k2, n = b.shape
    assert k == k2 and n % block_n == 0
    return pl.pallas_call(
        _mm_body,
        grid=(n // block_n,),
        in_specs=[
            pl.BlockSpec((m, k), lambda j: (0, 0)),
            pl.BlockSpec((k, block_n), lambda j: (0, j)),
        ],
        out_specs=pl.BlockSpec((m, block_n), lambda j: (0, j)),
        out_shape=jax.ShapeDtypeStruct((m, n), jnp.float32),
    )(a, b)


def _allreduce_mm_body(c_ref, w_ref, out_ref, recv_ref, send_sem, recv_sem):
    my_x = lax.axis_index("x")
    my_y = lax.axis_index("y")
    partner = (1 - my_x, my_y)

    out_ref[...] = jnp.dot(c_ref[...], w_ref[...],
                           preferred_element_type=jnp.float32)

    barrier = pltpu.get_barrier_semaphore()
    pl.semaphore_signal(barrier, inc=1, device_id=partner,
                        device_id_type=pl.DeviceIdType.MESH)
    pl.semaphore_wait(barrier, 1)

    rdma = pltpu.make_async_remote_copy(
        src_ref=out_ref,
        dst_ref=recv_ref,
        send_sem=send_sem,
        recv_sem=recv_sem,
        device_id=partner,
        device_id_type=pl.DeviceIdType.MESH,
    )
    rdma.start()
    rdma.wait()

    out_ref[...] = out_ref[...] + recv_ref[...]


def _allreduce_matmul(c, w, collective_id):
    m, k = c.shape
    k2, n = w.shape
    assert k == k2
    return pl.pallas_call(
        _allreduce_mm_body,
        in_specs=[
            pl.BlockSpec(memory_space=pltpu.VMEM),
            pl.BlockSpec(memory_space=pltpu.VMEM),
        ],
        out_specs=pl.BlockSpec(memory_space=pltpu.VMEM),
        out_shape=jax.ShapeDtypeStruct((m, n), jnp.float32),
        scratch_shapes=[
            pltpu.VMEM((m, n), jnp.float32),
            pltpu.SemaphoreType.DMA,
            pltpu.SemaphoreType.DMA,
        ],
        compiler_params=pltpu.CompilerParams(collective_id=collective_id),
    )(c, w)


_SCALE = (Dh + Dr) ** -0.5


def _attn_body(q_ref, qr_ref, k_ref, kr_ref, v_ref, o_ref):
    dn = (((1,), (1,)), ((), ()))
    s = lax.dot_general(q_ref[...], k_ref[...], dn,
                        preferred_element_type=jnp.float32)
    s = s + lax.dot_general(qr_ref[...], kr_ref[...], dn,
                            preferred_element_type=jnp.float32)
    s = s * _SCALE
    m = jnp.max(s, axis=-1, keepdims=True)
    p = jnp.exp(s - m)
    p = p / jnp.sum(p, axis=-1, keepdims=True)
    o_ref[...] = jnp.dot(p, v_ref[...], preferred_element_type=jnp.float32)


def _attention(q, qr, k, kr, v):
    return pl.pallas_call(
        _attn_body,
        grid=(B, H),
        in_specs=[
            pl.BlockSpec((S, Dh), lambda b, h: (b, h)),
            pl.BlockSpec((S, Dr), lambda b, h: (b, h)),
            pl.BlockSpec((S, Dh), lambda b, h: (b, h)),
            pl.BlockSpec((S, Dr), lambda b, h: (b, 0)),
            pl.BlockSpec((S, Dh), lambda b, h: (b, h)),
        ],
        out_specs=pl.BlockSpec((S, Dh), lambda b, h: (b, h)),
        out_shape=jax.ShapeDtypeStruct((M, H * Dh), jnp.float32),
    )(q, qr, k, kr, v)


def kernel(x, Wdkv, Wuk, Wuv, Wq, Wqr, Wkr, Wo):
    x2 = x.reshape(M, D)

    c = _matmul(x2, Wdkv, block_n=Wdkv.shape[1])

    k = _allreduce_matmul(c, Wuk, collective_id=0)
    v = _allreduce_matmul(c, Wuv, collective_id=1)

    q = _matmul(x2, Wq, block_n=512)
    qr = _matmul(x2, Wqr, block_n=512)
    kr = _matmul(x2, Wkr, block_n=Dr)

    o = _attention(q, qr, k, kr, v)
    out = _matmul(o, Wo, block_n=512)
    return out.reshape(B, S, D)


# baseline (device time: 673439 ns/iter reference)
import functools

import jax
import jax.numpy as jnp
from jax import lax
from jax.experimental import pallas as pl
from jax.experimental.pallas import tpu as pltpu

B, S, H, Dh, Dr = 4, 256, 32, 128, 64
D = 4096
DC = 256
M = B * S

_VMEM_LIMIT = 100 * 1024 * 1024


def _mm_body(a_ref, b_ref, o_ref):
    o_ref[...] = jnp.dot(a_ref[...], b_ref[...],
                         preferred_element_type=jnp.float32)


def _matmul(a, b, block_n):
    m, k = a.shape
    k2, n = b.shape
    assert k == k2 and n % block_n == 0
    return pl.pallas_call(
        _mm_body,
        grid=(n // block_n,),
        in_specs=[
            pl.BlockSpec((m, k), lambda j: (0, 0)),
            pl.BlockSpec((k, block_n), lambda j: (0, j)),
        ],
        out_specs=pl.BlockSpec((m, block_n), lambda j: (0, j)),
        out_shape=jax.ShapeDtypeStruct((m, n), jnp.float32),
        compiler_params=pltpu.CompilerParams(vmem_limit_bytes=_VMEM_LIMIT),
    )(a, b)


def _allreduce_mm_body(c_ref, w_ref, out_ref, recv_ref, send_sem, recv_sem):
    my_x = lax.axis_index("x")
    my_y = lax.axis_index("y")
    partner = (1 - my_x, my_y)

    out_ref[...] = jnp.dot(c_ref[...], w_ref[...],
                           preferred_element_type=jnp.float32)

    barrier = pltpu.get_barrier_semaphore()
    pl.semaphore_signal(barrier, inc=1, device_id=partner,
                        device_id_type=pl.DeviceIdType.MESH)
    pl.semaphore_wait(barrier, 1)

    rdma = pltpu.make_async_remote_copy(
        src_ref=out_ref,
        dst_ref=recv_ref,
        send_sem=send_sem,
        recv_sem=recv_sem,
        device_id=partner,
        device_id_type=pl.DeviceIdType.MESH,
    )
    rdma.start()
    rdma.wait()

    out_ref[...] = out_ref[...] + recv_ref[...]


def _allreduce_matmul(c, w, collective_id):
    m, k = c.shape
    k2, n = w.shape
    assert k == k2
    return pl.pallas_call(
        _allreduce_mm_body,
        in_specs=[
            pl.BlockSpec(memory_space=pltpu.VMEM),
            pl.BlockSpec(memory_space=pltpu.VMEM),
        ],
        out_specs=pl.BlockSpec(memory_space=pltpu.VMEM),
        out_shape=jax.ShapeDtypeStruct((m, n), jnp.float32),
        scratch_shapes=[
            pltpu.VMEM((m, n), jnp.float32),
            pltpu.SemaphoreType.DMA,
            pltpu.SemaphoreType.DMA,
        ],
        compiler_params=pltpu.CompilerParams(collective_id=collective_id,
                                             vmem_limit_bytes=_VMEM_LIMIT),
    )(c, w)


_SCALE = (Dh + Dr) ** -0.5


def _attn_body(q_ref, qr_ref, k_ref, kr_ref, v_ref, o_ref):
    dn = (((1,), (1,)), ((), ()))
    s = lax.dot_general(q_ref[...], k_ref[...], dn,
                        preferred_element_type=jnp.float32)
    s = s + lax.dot_general(qr_ref[...], kr_ref[...], dn,
                            preferred_element_type=jnp.float32)
    s = s * _SCALE
    m = jnp.max(s, axis=-1, keepdims=True)
    p = jnp.exp(s - m)
    p = p / jnp.sum(p, axis=-1, keepdims=True)
    o_ref[...] = jnp.dot(p, v_ref[...], preferred_element_type=jnp.float32)


def _attention(q, qr_pad, k, kr_pad, v):
    return pl.pallas_call(
        _attn_body,
        grid=(B, H),
        in_specs=[
            pl.BlockSpec((S, Dh), lambda b, h: (b, h)),
            pl.BlockSpec((S, 128), lambda b, h: (b, h)),
            pl.BlockSpec((S, Dh), lambda b, h: (b, h)),
            pl.BlockSpec((S, 128), lambda b, h: (b, 0)),
            pl.BlockSpec((S, Dh), lambda b, h: (b, h)),
        ],
        out_specs=pl.BlockSpec((S, Dh), lambda b, h: (b, h)),
        out_shape=jax.ShapeDtypeStruct((M, H * Dh), jnp.float32),
        compiler_params=pltpu.CompilerParams(vmem_limit_bytes=_VMEM_LIMIT),
    )(q, qr_pad, k, kr_pad, v)


def kernel(x, Wdkv, Wuk, Wuv, Wq, Wqr, Wkr, Wo):
    x2 = x.reshape(M, D)

    c = _matmul(x2, Wdkv, block_n=Wdkv.shape[1])

    k = _allreduce_matmul(c, Wuk, collective_id=0)
    v = _allreduce_matmul(c, Wuv, collective_id=1)

    q = _matmul(x2, Wq, block_n=512)
    qr = _matmul(x2, Wqr, block_n=512)
    kr = _matmul(x2, Wkr, block_n=Dr)

    qr_pad = jnp.pad(qr.reshape(M, H, Dr),
                     ((0, 0), (0, 0), (0, 128 - Dr))).reshape(M, H * 128)
    kr_pad = jnp.pad(kr, ((0, 0), (0, 128 - Dr)))

    o = _attention(q, qr_pad, k, kr_pad, v)
    out = _matmul(o, Wo, block_n=512)
    return out.reshape(B, S, D)


# device time: 425237 ns/iter; 1.5837x vs baseline; 1.5837x over previous
import jax
import jax.numpy as jnp
from jax import lax
from jax.experimental import pallas as pl
from jax.experimental.pallas import tpu as pltpu

B, S, H, Dh, Dr = 4, 256, 32, 128, 64
D = 4096
M = B * S
NH = H // 2
QH = NH * Dh
QRH = NH * Dr

_VMEM_LIMIT = 100 * 1024 * 1024
_MESH = pl.DeviceIdType.MESH
_F32 = jnp.float32
_BF16 = jnp.bfloat16


def _mm_body(a_ref, b_ref, o_ref):
    o_ref[...] = jnp.dot(a_ref[...], b_ref[...], preferred_element_type=_F32)


def _matmul(a, b, block_n):
    m, k = a.shape
    k2, n = b.shape
    assert k == k2 and n % block_n == 0
    return pl.pallas_call(
        _mm_body,
        grid=(n // block_n,),
        in_specs=[
            pl.BlockSpec((m, k), lambda j: (0, 0)),
            pl.BlockSpec((k, block_n), lambda j: (0, j)),
        ],
        out_specs=pl.BlockSpec((m, block_n), lambda j: (0, j)),
        out_shape=jax.ShapeDtypeStruct((m, n), _F32),
        compiler_params=pltpu.CompilerParams(vmem_limit_bytes=_VMEM_LIMIT),
    )(a, b)


_KV_STEPS = 8


def _proj_body(c_ref, wuk_ref, wuv_ref, x_ref, wq_ref,
               k_ref, v_ref, q_ref,
               sk, sv, rk, rv, sem_sk, sem_rk, sem_sv, sem_rv):
    j = pl.program_id(0)
    my_x = lax.axis_index("x")
    my_y = lax.axis_index("y")
    partner = (1 - my_x, my_y)

    def rdma_k():
        return pltpu.make_async_remote_copy(
            src_ref=sk, dst_ref=rk, send_sem=sem_sk, recv_sem=sem_rk,
            device_id=partner, device_id_type=_MESH)

    def rdma_v():
        return pltpu.make_async_remote_copy(
            src_ref=sv, dst_ref=rv, send_sem=sem_sv, recv_sem=sem_rv,
            device_id=partner, device_id_type=_MESH)

    @pl.when(j == 0)
    def _():
        kp = jnp.dot(c_ref[...], wuk_ref[...], preferred_element_type=_F32)
        vp = jnp.dot(c_ref[...], wuv_ref[...], preferred_element_type=_F32)
        k_ref[...] = kp
        v_ref[...] = vp
        sk[...] = kp.astype(_BF16)
        sv[...] = vp.astype(_BF16)
        barrier = pltpu.get_barrier_semaphore()
        pl.semaphore_signal(barrier, inc=1, device_id=partner,
                            device_id_type=_MESH)
        pl.semaphore_wait(barrier, 1)
        rdma_k().start()
        rdma_v().start()

    q_ref[...] = jnp.dot(x_ref[...], wq_ref[...], preferred_element_type=_F32)

    @pl.when(j == _KV_STEPS - 1)
    def _():
        rdma_k().wait()
        k_ref[...] = k_ref[...] + rk[...].astype(_F32)
        rdma_v().wait()
        v_ref[...] = v_ref[...] + rv[...].astype(_F32)


def _proj_kv_allreduce(c, wuk_h, wuv_h, x2, wq_h):
    bq = QH // _KV_STEPS
    return pl.pallas_call(
        _proj_body,
        grid=(_KV_STEPS,),
        in_specs=[
            pl.BlockSpec((M, c.shape[1]), lambda j: (0, 0)),
            pl.BlockSpec(wuk_h.shape, lambda j: (0, 0)),
            pl.BlockSpec(wuv_h.shape, lambda j: (0, 0)),
            pl.BlockSpec((M, D), lambda j: (0, 0)),
            pl.BlockSpec((D, bq), lambda j: (0, j)),
        ],
        out_specs=[
            pl.BlockSpec((M, QH), lambda j: (0, 0)),
            pl.BlockSpec((M, QH), lambda j: (0, 0)),
            pl.BlockSpec((M, bq), lambda j: (0, j)),
        ],
        out_shape=[
            jax.ShapeDtypeStruct((M, QH), _F32),
            jax.ShapeDtypeStruct((M, QH), _F32),
            jax.ShapeDtypeStruct((M, QH), _F32),
        ],
        scratch_shapes=[
            pltpu.VMEM((M, QH), _BF16),
            pltpu.VMEM((M, QH), _BF16),
            pltpu.VMEM((M, QH), _BF16),
            pltpu.VMEM((M, QH), _BF16),
            pltpu.SemaphoreType.DMA,
            pltpu.SemaphoreType.DMA,
            pltpu.SemaphoreType.DMA,
            pltpu.SemaphoreType.DMA,
        ],
        compiler_params=pltpu.CompilerParams(collective_id=0,
                                             vmem_limit_bytes=_VMEM_LIMIT),
    )(c, wuk_h, wuv_h, x2, wq_h)


_SCALE = (Dh + Dr) ** -0.5


def _attn_body(q_ref, qr_ref, k_ref, kr_ref, v_ref, o_ref):
    dn = (((1,), (1,)), ((), ()))
    s = lax.dot_general(q_ref[...], k_ref[...], dn,
                        preferred_element_type=_F32)
    s = s + lax.dot_general(qr_ref[...], kr_ref[...], dn,
                            preferred_element_type=_F32)
    s = s * _SCALE
    m = jnp.max(s, axis=-1, keepdims=True)
    p = jnp.exp(s - m)
    p = p / jnp.sum(p, axis=-1, keepdims=True)
    o_ref[...] = jnp.dot(p, v_ref[...], preferred_element_type=_F32)


def _attention(q, qr_pad, k, kr_pad, v):
    return pl.pallas_call(
        _attn_body,
        grid=(B, NH),
        in_specs=[
            pl.BlockSpec((S, Dh), lambda b, h: (b, h)),
            pl.BlockSpec((S, 128), lambda b, h: (b, h)),
            pl.BlockSpec((S, Dh), lambda b, h: (b, h)),
            pl.BlockSpec((S, 128), lambda b, h: (b, 0)),
            pl.BlockSpec((S, Dh), lambda b, h: (b, h)),
        ],
        out_specs=pl.BlockSpec((S, Dh), lambda b, h: (b, h)),
        out_shape=jax.ShapeDtypeStruct((M, QH), _F32),
        compiler_params=pltpu.CompilerParams(vmem_limit_bytes=_VMEM_LIMIT),
    )(q, qr_pad, k, kr_pad, v)


_OP_STEPS = 5


def _outproj_body(o_ref, wo_ref, out_ref, snd, rcv, sem_s, sem_r):
    j = pl.program_id(0)
    my_x = lax.axis_index("x")
    my_y = lax.axis_index("y")
    partner = (my_x, 1 - my_y)

    def rdma():
        return pltpu.make_async_remote_copy(
            src_ref=snd, dst_ref=rcv, send_sem=sem_s, recv_sem=sem_r,
            device_id=partner, device_id_type=_MESH)

    @pl.when(j == 0)
    def _():
        out_ref[...] = jnp.dot(o_ref[...], wo_ref[...],
                               preferred_element_type=_F32)

    @pl.when((j > 0) & (j < _OP_STEPS - 1))
    def _():
        out_ref[...] = out_ref[...] + jnp.dot(
            o_ref[...], wo_ref[...], preferred_element_type=_F32)

    @pl.when(j == _OP_STEPS - 1)
    def _():
        snd[...] = out_ref[...].astype(_BF16)
        barrier = pltpu.get_barrier_semaphore()
        pl.semaphore_signal(barrier, inc=1, device_id=partner,
                            device_id_type=_MESH)
        pl.semaphore_wait(barrier, 1)
        r = rdma()
        r.start()
        r.wait()
        out_ref[...] = out_ref[...] + rcv[...].astype(_F32)


def _outproj_allreduce(o_h, wo_h):
    bk = QH // (_OP_STEPS - 1)
    return pl.pallas_call(
        _outproj_body,
        grid=(_OP_STEPS,),
        in_specs=[
            pl.BlockSpec((M, bk), lambda j: (0, jnp.minimum(j, _OP_STEPS - 2))),
            pl.BlockSpec((bk, D), lambda j: (jnp.minimum(j, _OP_STEPS - 2), 0)),
        ],
        out_specs=pl.BlockSpec((M, D), lambda j: (0, 0)),
        out_shape=jax.ShapeDtypeStruct((M, D), _F32),
        scratch_shapes=[
            pltpu.VMEM((M, D), _BF16),
            pltpu.VMEM((M, D), _BF16),
            pltpu.SemaphoreType.DMA,
            pltpu.SemaphoreType.DMA,
        ],
        compiler_params=pltpu.CompilerParams(collective_id=1,
                                             vmem_limit_bytes=_VMEM_LIMIT),
    )(o_h, wo_h)


def kernel(x, Wdkv, Wuk, Wuv, Wq, Wqr, Wkr, Wo):
    x2 = x.reshape(M, D)
    my_y = lax.axis_index("y")

    c = _matmul(x2, Wdkv, block_n=Wdkv.shape[1])

    wuk_h = lax.dynamic_slice(Wuk, (0, my_y * QH), (Wuk.shape[0], QH))
    wuv_h = lax.dynamic_slice(Wuv, (0, my_y * QH), (Wuv.shape[0], QH))
    wq_h = lax.dynamic_slice(Wq, (0, my_y * QH), (D, QH))
    wqr_h = lax.dynamic_slice(Wqr, (0, my_y * QRH), (D, QRH))
    wo_h = lax.dynamic_slice(Wo, (my_y * QH, 0), (QH, D))

    k_h, v_h, q_h = _proj_kv_allreduce(c, wuk_h, wuv_h, x2, wq_h)
    qr_h = _matmul(x2, wqr_h, block_n=256)
    kr = _matmul(x2, Wkr, block_n=Dr)

    qr_pad = jnp.pad(qr_h.reshape(M, NH, Dr),
                     ((0, 0), (0, 0), (0, 128 - Dr))).reshape(M, NH * 128)
    kr_pad = jnp.pad(kr, ((0, 0), (0, 128 - Dr)))

    o_h = _attention(q_h, qr_pad, k_h, kr_pad, v_h)

    out = _outproj_allreduce(o_h, wo_h)
    return out.reshape(B, S, D)


# device time: 360768 ns/iter; 1.8667x vs baseline; 1.1787x over previous
import jax
import jax.numpy as jnp
from jax import lax
from jax.experimental import pallas as pl
from jax.experimental.pallas import tpu as pltpu

B, S, H, Dh, Dr = 4, 256, 32, 128, 64
D = 4096
M = B * S
NH = H // 2
QH = NH * Dh
QRH = NH * Dr

_VMEM_LIMIT = 100 * 1024 * 1024
_MESH = pl.DeviceIdType.MESH
_F32 = jnp.float32
_BF16 = jnp.bfloat16


def _sel(i):
    return jnp.reshape(i, (1,)).astype(jnp.int32)


def _mm_body(a_ref, b_ref, o_ref):
    o_ref[...] = jnp.dot(a_ref[...], b_ref[...], preferred_element_type=_F32)


def _matmul(a, b, block_n):
    m, k = a.shape
    k2, n = b.shape
    assert k == k2 and n % block_n == 0
    return pl.pallas_call(
        _mm_body,
        grid=(n // block_n,),
        in_specs=[
            pl.BlockSpec((m, k), lambda j: (0, 0)),
            pl.BlockSpec((k, block_n), lambda j: (0, j)),
        ],
        out_specs=pl.BlockSpec((m, block_n), lambda j: (0, j)),
        out_shape=jax.ShapeDtypeStruct((m, n), _F32),
        compiler_params=pltpu.CompilerParams(vmem_limit_bytes=_VMEM_LIMIT),
    )(a, b)


def _mm_half_body(s_ref, a_ref, b_ref, o_ref):
    o_ref[...] = jnp.dot(a_ref[...], b_ref[...], preferred_element_type=_F32)


def _matmul_half(a, b, sel, n_half, block_n):
    m, k = a.shape
    nblk = n_half // block_n
    grid_spec = pltpu.PrefetchScalarGridSpec(
        num_scalar_prefetch=1,
        grid=(nblk,),
        in_specs=[
            pl.BlockSpec((m, k), lambda j, s: (0, 0)),
            pl.BlockSpec((k, block_n), lambda j, s: (0, s[0] * nblk + j)),
        ],
        out_specs=pl.BlockSpec((m, block_n), lambda j, s: (0, j)),
    )
    return pl.pallas_call(
        _mm_half_body,
        grid_spec=grid_spec,
        out_shape=jax.ShapeDtypeStruct((m, n_half), _F32),
        compiler_params=pltpu.CompilerParams(vmem_limit_bytes=_VMEM_LIMIT),
    )(_sel(sel), a, b)


_KV_STEPS = 8


def _proj_body(s_ref, c_ref, wuk_ref, wuv_ref, x_ref, wq_ref,
               k_ref, v_ref, q_ref,
               sk, sv, rk, rv, sem_sk, sem_rk, sem_sv, sem_rv):
    j = pl.program_id(0)
    my_x = lax.axis_index("x")
    my_y = lax.axis_index("y")
    partner = (1 - my_x, my_y)

    def rdma_k():
        return pltpu.make_async_remote_copy(
            src_ref=sk, dst_ref=rk, send_sem=sem_sk, recv_sem=sem_rk,
            device_id=partner, device_id_type=_MESH)

    def rdma_v():
        return pltpu.make_async_remote_copy(
            src_ref=sv, dst_ref=rv, send_sem=sem_sv, recv_sem=sem_rv,
            device_id=partner, device_id_type=_MESH)

    @pl.when(j == 0)
    def _():
        kp = jnp.dot(c_ref[...], wuk_ref[...], preferred_element_type=_F32)
        vp = jnp.dot(c_ref[...], wuv_ref[...], preferred_element_type=_F32)
        k_ref[...] = kp
        v_ref[...] = vp
        sk[...] = kp.astype(_BF16)
        sv[...] = vp.astype(_BF16)
        barrier = pltpu.get_barrier_semaphore()
        pl.semaphore_signal(barrier, inc=1, device_id=partner,
                            device_id_type=_MESH)
        pl.semaphore_wait(barrier, 1)
        rdma_k().start()
        rdma_v().start()

    q_ref[...] = jnp.dot(x_ref[...], wq_ref[...], preferred_element_type=_F32)

    @pl.when(j == _KV_STEPS - 1)
    def _():
        rdma_k().wait()
        k_ref[...] = k_ref[...] + rk[...].astype(_F32)
        rdma_v().wait()
        v_ref[...] = v_ref[...] + rv[...].astype(_F32)


def _proj_kv_allreduce(my_y, c, wuk, wuv, x2, wq):
    bq = QH // _KV_STEPS
    dc = c.shape[1]
    grid_spec = pltpu.PrefetchScalarGridSpec(
        num_scalar_prefetch=1,
        grid=(_KV_STEPS,),
        in_specs=[
            pl.BlockSpec((M, dc), lambda j, s: (0, 0)),
            pl.BlockSpec((dc, QH), lambda j, s: (0, s[0])),
            pl.BlockSpec((dc, QH), lambda j, s: (0, s[0])),
            pl.BlockSpec((M, D), lambda j, s: (0, 0)),
            pl.BlockSpec((D, bq), lambda j, s: (0, s[0] * _KV_STEPS + j)),
        ],
        out_specs=[
            pl.BlockSpec((M, QH), lambda j, s: (0, 0)),
            pl.BlockSpec((M, QH), lambda j, s: (0, 0)),
            pl.BlockSpec((M, bq), lambda j, s: (0, j)),
        ],
        scratch_shapes=[
            pltpu.VMEM((M, QH), _BF16),
            pltpu.VMEM((M, QH), _BF16),
            pltpu.VMEM((M, QH), _BF16),
            pltpu.VMEM((M, QH), _BF16),
            pltpu.SemaphoreType.DMA,
            pltpu.SemaphoreType.DMA,
            pltpu.SemaphoreType.DMA,
            pltpu.SemaphoreType.DMA,
        ],
    )
    return pl.pallas_call(
        _proj_body,
        grid_spec=grid_spec,
        out_shape=[
            jax.ShapeDtypeStruct((M, QH), _F32),
            jax.ShapeDtypeStruct((M, QH), _F32),
            jax.ShapeDtypeStruct((M, QH), _F32),
        ],
        compiler_params=pltpu.CompilerParams(collective_id=0,
                                             vmem_limit_bytes=_VMEM_LIMIT),
    )(_sel(my_y), c, wuk, wuv, x2, wq)


_SCALE = (Dh + Dr) ** -0.5


def _attn_body(q_ref, qr_ref, k_ref, kr_ref, v_ref, o_ref):
    dn = (((1,), (1,)), ((), ()))
    s = lax.dot_general(q_ref[...], k_ref[...], dn,
                        preferred_element_type=_F32)
    s = s + lax.dot_general(qr_ref[...], kr_ref[...], dn,
                            preferred_element_type=_F32)
    s = s * _SCALE
    m = jnp.max(s, axis=-1, keepdims=True)
    p = jnp.exp(s - m)
    p = p / jnp.sum(p, axis=-1, keepdims=True)
    o_ref[...] = jnp.dot(p, v_ref[...], preferred_element_type=_F32)


def _attention(q, qr_pad, k, kr_pad, v):
    return pl.pallas_call(
        _attn_body,
        grid=(B, NH),
        in_specs=[
            pl.BlockSpec((S, Dh), lambda b, h: (b, h)),
            pl.BlockSpec((S, 128), lambda b, h: (b, h)),
            pl.BlockSpec((S, Dh), lambda b, h: (b, h)),
            pl.BlockSpec((S, 128), lambda b, h: (b, 0)),
            pl.BlockSpec((S, Dh), lambda b, h: (b, h)),
        ],
        out_specs=pl.BlockSpec((S, Dh), lambda b, h: (b, h)),
        out_shape=jax.ShapeDtypeStruct((M, QH), _F32),
        compiler_params=pltpu.CompilerParams(vmem_limit_bytes=_VMEM_LIMIT),
    )(q, qr_pad, k, kr_pad, v)


_OP_STEPS = 5


def _outproj_body(s_ref, o_ref, wo_ref, out_ref, snd, rcv, sem_s, sem_r):
    j = pl.program_id(0)
    my_x = lax.axis_index("x")
    my_y = lax.axis_index("y")
    partner = (my_x, 1 - my_y)

    def rdma():
        return pltpu.make_async_remote_copy(
            src_ref=snd, dst_ref=rcv, send_sem=sem_s, recv_sem=sem_r,
            device_id=partner, device_id_type=_MESH)

    @pl.when(j == 0)
    def _():
        out_ref[...] = jnp.dot(o_ref[...], wo_ref[...],
                               preferred_element_type=_F32)

    @pl.when((j > 0) & (j < _OP_STEPS - 1))
    def _():
        out_ref[...] = out_ref[...] + jnp.dot(
            o_ref[...], wo_ref[...], preferred_element_type=_F32)

    @pl.when(j == _OP_STEPS - 1)
    def _():
        snd[...] = out_ref[...].astype(_BF16)
        barrier = pltpu.get_barrier_semaphore()
        pl.semaphore_signal(barrier, inc=1, device_id=partner,
                            device_id_type=_MESH)
        pl.semaphore_wait(barrier, 1)
        r = rdma()
        r.start()
        r.wait()
        out_ref[...] = out_ref[...] + rcv[...].astype(_F32)


def _outproj_allreduce(my_y, o_h, wo):
    nk = _OP_STEPS - 1
    bk = QH // nk
    grid_spec = pltpu.PrefetchScalarGridSpec(
        num_scalar_prefetch=1,
        grid=(_OP_STEPS,),
        in_specs=[
            pl.BlockSpec((M, bk),
                         lambda j, s: (0, jnp.minimum(j, nk - 1))),
            pl.BlockSpec((bk, D),
                         lambda j, s: (s[0] * nk + jnp.minimum(j, nk - 1), 0)),
        ],
        out_specs=pl.BlockSpec((M, D), lambda j, s: (0, 0)),
        scratch_shapes=[
            pltpu.VMEM((M, D), _BF16),
            pltpu.VMEM((M, D), _BF16),
            pltpu.SemaphoreType.DMA,
            pltpu.SemaphoreType.DMA,
        ],
    )
    return pl.pallas_call(
        _outproj_body,
        grid_spec=grid_spec,
        out_shape=jax.ShapeDtypeStruct((M, D), _F32),
        compiler_params=pltpu.CompilerParams(collective_id=1,
                                             vmem_limit_bytes=_VMEM_LIMIT),
    )(_sel(my_y), o_h, wo)


def kernel(x, Wdkv, Wuk, Wuv, Wq, Wqr, Wkr, Wo):
    x2 = x.reshape(M, D)
    my_y = lax.axis_index("y")

    c = _matmul(x2, Wdkv, block_n=Wdkv.shape[1])

    k_h, v_h, q_h = _proj_kv_allreduce(my_y, c, Wuk, Wuv, x2, Wq)
    qr_h = _matmul_half(x2, Wqr, my_y, QRH, block_n=256)
    kr = _matmul(x2, Wkr, block_n=Dr)

    qr_pad = jnp.pad(qr_h.reshape(M, NH, Dr),
                     ((0, 0), (0, 0), (0, 128 - Dr))).reshape(M, NH * 128)
    kr_pad = jnp.pad(kr, ((0, 0), (0, 128 - Dr)))

    o_h = _attention(q_h, qr_pad, k_h, kr_pad, v_h)

    out = _outproj_allreduce(my_y, o_h, Wo)
    return out.reshape(B, S, D)
